# Initial kernel scaffold; baseline (speedup 1.0000x reference)
import jax
import jax.numpy as jnp
from jax.experimental import pallas as pl


def kernel(pred, target):
    raise NotImplementedError("write your pallas kernel here")


# speedup vs baseline: 1.5447x; 1.5447x over previous
import functools

import jax
import jax.numpy as jnp
from jax.experimental import pallas as pl
from jax.experimental.pallas import tpu as pltpu

N_BLK = 256
M_CHUNK = 512


def _emd_kernel(p_ref, t_ref, o_ref, *, m_total):
    p = p_ref[0]
    t3 = t_ref[0]
    ta = jnp.concatenate(
        [t3 * (-2.0), t3 * t3, jnp.zeros((2, m_total), jnp.float32)], axis=0
    )

    acc = None
    for j in range(m_total // M_CHUNK):
        z = jax.lax.dot_general(
            p, ta[:, j * M_CHUNK:(j + 1) * M_CHUNK],
            (((1,), (0,)), ((), ())),
            preferred_element_type=jnp.float32,
        )
        for l in range(M_CHUNK // 128):
            tile = z[:, l * 128:(l + 1) * 128]
            acc = tile if acc is None else jnp.minimum(acc, tile)

    zmin = jnp.min(acc, axis=1, keepdims=True)
    p2 = jnp.sum(p * p, axis=1, keepdims=True) - 3.0
    d2 = jnp.maximum(zmin + p2, 0.0)
    o_ref[0] = jnp.sqrt(d2)


def kernel(pred, target):
    B, N, _ = pred.shape
    M = target.shape[1]

    ones = jnp.ones((B, N, 3), jnp.float32)
    zeros = jnp.zeros((B, N, 2), jnp.float32)
    pred_aug = jnp.concatenate([pred, ones, zeros], axis=-1)
    target_t = jnp.transpose(target, (0, 2, 1))

    grid = (B, N // N_BLK)
    min_d = pl.pallas_call(
        functools.partial(_emd_kernel, m_total=M),
        grid=grid,
        in_specs=[
            pl.BlockSpec((1, N_BLK, 8), lambda b, n: (b, n, 0)),
            pl.BlockSpec((1, 3, M), lambda b, n: (b, 0, 0)),
        ],
        out_specs=pl.BlockSpec((1, N_BLK, 1), lambda b, n: (b, n, 0)),
        out_shape=jax.ShapeDtypeStruct((B, N, 1), jnp.float32),
        compiler_params=pltpu.CompilerParams(
            dimension_semantics=("parallel", "parallel"),
        ),
    )(pred_aug, target_t)

    return jnp.mean(min_d[..., 0])


# speedup vs baseline: 2.1654x; 1.4019x over previous
import functools

import jax
import jax.numpy as jnp
from jax.experimental import pallas as pl
from jax.experimental.pallas import tpu as pltpu

N_SUB = 256
SUBS = 8
M_CHUNK = 512


def _emd_kernel(p_ref, t_ref, o_ref, *, m_total):
    t3 = t_ref[0]
    ta = jnp.concatenate(
        [t3 * (-2.0), t3 * t3, jnp.zeros((2, m_total), jnp.float32)], axis=0
    )

    for s in range(SUBS):
        p = p_ref[0, s * N_SUB:(s + 1) * N_SUB, :]
        acc = None
        for j in range(m_total // M_CHUNK):
            z = jax.lax.dot_general(
                p, ta[:, j * M_CHUNK:(j + 1) * M_CHUNK],
                (((1,), (0,)), ((), ())),
                preferred_element_type=jnp.float32,
            )
            for l in range(M_CHUNK // 128):
                tile = z[:, l * 128:(l + 1) * 128]
                acc = tile if acc is None else jnp.minimum(acc, tile)

        zmin = jnp.min(acc, axis=1, keepdims=True)
        p2 = jnp.sum(p * p, axis=1, keepdims=True) - 3.0
        d2 = jnp.maximum(zmin + p2, 0.0)
        o_ref[0, s * N_SUB:(s + 1) * N_SUB, :] = jnp.sqrt(d2)


def kernel(pred, target):
    B, N, _ = pred.shape
    M = target.shape[1]
    n_blk = N_SUB * SUBS

    ones = jnp.ones((B, N, 3), jnp.float32)
    zeros = jnp.zeros((B, N, 2), jnp.float32)
    pred_aug = jnp.concatenate([pred, ones, zeros], axis=-1)
    target_t = jnp.transpose(target, (0, 2, 1))

    grid = (B, N // n_blk)
    min_d = pl.pallas_call(
        functools.partial(_emd_kernel, m_total=M),
        grid=grid,
        in_specs=[
            pl.BlockSpec((1, n_blk, 8), lambda b, n: (b, n, 0)),
            pl.BlockSpec((1, 3, M), lambda b, n: (b, 0, 0)),
        ],
        out_specs=pl.BlockSpec((1, n_blk, 1), lambda b, n: (b, n, 0)),
        out_shape=jax.ShapeDtypeStruct((B, N, 1), jnp.float32),
        compiler_params=pltpu.CompilerParams(
            dimension_semantics=("parallel", "arbitrary"),
        ),
    )(pred_aug, target_t)

    return jnp.mean(min_d[..., 0])
